# Initial kernel scaffold; baseline (speedup 1.0000x reference)
import jax
import jax.numpy as jnp
from jax.experimental import pallas as pl


def kernel(input, int_table, float_table):
    raise NotImplementedError("write your pallas kernel here")


# speedup vs baseline: 1.2932x; 1.2932x over previous
import functools

import jax
import jax.numpy as jnp
from jax import lax
from jax.experimental import pallas as pl
from jax.experimental.pallas import tpu as pltpu
from jax.experimental.pallas import tpu_sc as plsc

_VOCAB = 1000000
_HID = 32
_B = 4096
_L = 50
_N = _B * _L

_NC = 2
_NS = 16
_NW = _NC * _NS
_PER_W = _N // _NW
_CH = 128
_NCH = _PER_W // _CH
_LANES = 16


def _sc_body(inp_hbm, int_tab_hbm, ft_hbm, out_hbm,
             x_v, ii_v, fi_v, rows_i, rows_f, sem_i, sem_f):
    wid = lax.axis_index("s") * _NC + lax.axis_index("c")
    base = wid * _PER_W

    pltpu.sync_copy(inp_hbm.at[pl.ds(base, _PER_W)], x_v)

    def idx_chunk(c, _):
        for k in range(_CH // _LANES):
            x = x_v[pl.ds(c * _CH + k * _LANES, _LANES)]
            ii = x.astype(jnp.int32)
            fr = x - ii.astype(jnp.float32)
            fi = (fr * 100.0).astype(jnp.int32)
            ii_v[c, pl.ds(k * _LANES, _LANES)] = ii
            fi_v[c, pl.ds(k * _LANES, _LANES)] = fi
        return 0

    lax.fori_loop(0, _NCH, idx_chunk, 0)

    def gather_chunk(c, _):
        cp_i = pltpu.async_copy(int_tab_hbm.at[ii_v.at[c]], rows_i, sem_i)
        cp_f = pltpu.async_copy(ft_hbm.at[fi_v.at[c]], rows_f, sem_f)
        cp_i.wait()
        cp_f.wait()

        def add_row(j, _):
            for h in range(_HID // _LANES):
                a = rows_i[j, pl.ds(h * _LANES, _LANES)]
                b = rows_f[j, pl.ds(h * _LANES, _LANES)]
                rows_i[j, pl.ds(h * _LANES, _LANES)] = a + b
            return 0

        lax.fori_loop(0, _CH, add_row, 0)
        pltpu.sync_copy(rows_i, out_hbm.at[pl.ds(base + c * _CH, _CH)])
        return 0

    lax.fori_loop(0, _NCH, gather_chunk, 0)


@functools.partial(jax.jit)
def kernel(input, int_table, float_table):
    mesh = plsc.VectorSubcoreMesh(core_axis_name="c", subcore_axis_name="s")
    flat = input.reshape(_N)
    sc_call = pl.kernel(
        _sc_body,
        out_type=jax.ShapeDtypeStruct((_N, _HID), jnp.float32),
        mesh=mesh,
        compiler_params=pltpu.CompilerParams(use_tc_tiling_on_sc=False),
        scratch_types=[
            pltpu.VMEM((_PER_W,), jnp.float32),
            pltpu.VMEM((_NCH, _CH), jnp.int32),
            pltpu.VMEM((_NCH, _CH), jnp.int32),
            pltpu.VMEM((_CH, _HID), jnp.float32),
            pltpu.VMEM((_CH, _HID), jnp.float32),
            pltpu.SemaphoreType.DMA,
            pltpu.SemaphoreType.DMA,
        ],
    )
    out_flat = sc_call(flat, int_table, float_table)
    return out_flat.reshape(_B, _L, _HID)


# speedup vs baseline: 1.5451x; 1.1948x over previous
import functools

import jax
import jax.numpy as jnp
from jax import lax
from jax.experimental import pallas as pl
from jax.experimental import layout as jex_layout
from jax.experimental.pallas import tpu as pltpu
from jax.experimental.pallas import tpu_sc as plsc

_VOCAB = 1000000
_HID = 32
_B = 4096
_L = 50
_N = _B * _L

_NC = 2
_NS = 16
_NW = _NC * _NS
_PER_W = _N // _NW
_CH = 128
_NCH = _PER_W // _CH
_LANES = 16


def _sc_body(inp_hbm, tab_hbm, ft_hbm, out_hbm,
             x_v, ii_v, fi_v, rows_ia, rows_ib, rows_fa, rows_fb,
             out_a, out_b,
             sem_gia, sem_gib, sem_gfa, sem_gfb, sem_oa, sem_ob):
    wid = lax.axis_index("s") * _NC + lax.axis_index("c")
    base = wid * _PER_W

    pltpu.sync_copy(inp_hbm.at[pl.ds(base, _PER_W)], x_v)

    def idx_chunk(c, _):
        for k in range(_CH // _LANES):
            x = x_v[pl.ds(c * _CH + k * _LANES, _LANES)]
            ii = x.astype(jnp.int32)
            fr = x - ii.astype(jnp.float32)
            fi = (fr * 100.0).astype(jnp.int32)
            ii_v[c, pl.ds(k * _LANES, _LANES)] = ii
            fi_v[c, pl.ds(k * _LANES, _LANES)] = fi
        return 0

    lax.fori_loop(0, _NCH, idx_chunk, 0)

    def issue_gathers(c, rows_i, rows_f, sem_i, sem_f):
        pltpu.async_copy(tab_hbm.at[ii_v.at[c]], rows_i, sem_i)
        pltpu.async_copy(ft_hbm.at[fi_v.at[c]], rows_f, sem_f)

    def wait_gathers(c, rows_i, rows_f, sem_i, sem_f):
        pltpu.make_async_copy(tab_hbm.at[ii_v.at[c]], rows_i, sem_i).wait()
        pltpu.make_async_copy(ft_hbm.at[fi_v.at[c]], rows_f, sem_f).wait()

    def add_chunk(rows_i, rows_f, outb):
        def add_rows(j, _):
            for u in range(4):
                jj = j * 4 + u
                for h in range(_HID // _LANES):
                    a = rows_i[jj, pl.ds(h * _LANES, _LANES)]
                    b = rows_f[jj, pl.ds(h * _LANES, _LANES)]
                    outb[jj, pl.ds(h * _LANES, _LANES)] = a + b
            return 0

        lax.fori_loop(0, _CH // 4, add_rows, 0)

    def issue_store(c, outb, sem):
        pltpu.async_copy(outb, out_hbm.at[pl.ds(base + c * _CH, _CH)], sem)

    def wait_store(c, outb, sem):
        pltpu.make_async_copy(outb, out_hbm.at[pl.ds(base + c * _CH, _CH)], sem).wait()

    issue_gathers(0, rows_ia, rows_fa, sem_gia, sem_gfa)

    def pair(cp, _):
        c0 = cp * 2
        c1 = c0 + 1
        issue_gathers(c1, rows_ib, rows_fb, sem_gib, sem_gfb)

        @pl.when(cp > 0)
        def _():
            wait_store(c0 - 2, out_a, sem_oa)

        wait_gathers(c0, rows_ia, rows_fa, sem_gia, sem_gfa)
        add_chunk(rows_ia, rows_fa, out_a)
        issue_store(c0, out_a, sem_oa)

        @pl.when(cp < _NCH // 2 - 1)
        def _():
            issue_gathers(c0 + 2, rows_ia, rows_fa, sem_gia, sem_gfa)

        @pl.when(cp > 0)
        def _():
            wait_store(c1 - 2, out_b, sem_ob)

        wait_gathers(c1, rows_ib, rows_fb, sem_gib, sem_gfb)
        add_chunk(rows_ib, rows_fb, out_b)
        issue_store(c1, out_b, sem_ob)
        return 0

    lax.fori_loop(0, _NCH // 2, pair, 0)
    wait_store(_NCH - 2, out_a, sem_oa)
    wait_store(_NCH - 1, out_b, sem_ob)


@functools.partial(jax.jit)
def kernel(input, int_table, float_table):
    mesh = plsc.VectorSubcoreMesh(core_axis_name="c", subcore_axis_name="s")
    flat = input.reshape(_N)
    sc_call = pl.kernel(
        _sc_body,
        out_type=jax.ShapeDtypeStruct((_N, _HID), jnp.float32),
        mesh=mesh,
        compiler_params=pltpu.CompilerParams(use_tc_tiling_on_sc=False),
        scratch_types=[
            pltpu.VMEM((_PER_W,), jnp.float32),
            pltpu.VMEM((_NCH, _CH), jnp.int32),
            pltpu.VMEM((_NCH, _CH), jnp.int32),
            pltpu.VMEM((_CH, _HID), jnp.float32),
            pltpu.VMEM((_CH, _HID), jnp.float32),
            pltpu.VMEM((_CH, _HID), jnp.float32),
            pltpu.VMEM((_CH, _HID), jnp.float32),
            pltpu.VMEM((_CH, _HID), jnp.float32),
            pltpu.VMEM((_CH, _HID), jnp.float32),
            pltpu.SemaphoreType.DMA,
            pltpu.SemaphoreType.DMA,
            pltpu.SemaphoreType.DMA,
            pltpu.SemaphoreType.DMA,
            pltpu.SemaphoreType.DMA,
            pltpu.SemaphoreType.DMA,
        ],
    )
    out_flat = sc_call(flat, int_table, float_table)
    out = out_flat.reshape(_B, _L, _HID)
    return jex_layout.with_layout_constraint(out, jex_layout.Layout((0, 1, 2)))
